# baseline (speedup 1.0000x reference)
import jax
import jax.numpy as jnp
from jax import lax
from jax.experimental import pallas as pl
from jax.experimental.pallas import tpu as pltpu
from jax.experimental.pallas import tpu_sc as plsc

_ROWS = 128
_N = 32768
_K = 64
_NC = 2
_NS = 16
_L = 16
_ROWS_PER_W = _ROWS // (_NC * _NS)

_NVEC = _N // _L
_NG = 4
_COLCAP = _NVEC // _NG + 1

_NEG_INF = float("-inf")


def _splat_f32(v):
    return jnp.full((_L,), v, dtype=jnp.float32)


def _sort_desc(v):
    k, _ = plsc.sort_key_val(v, v, descending=True)
    return k


def _merge_cascade(c_sorted, s_regs):
    out = []
    carry = c_sorted
    for s in s_regs:
        r = lax.rev(carry, (0,))
        hi = jnp.maximum(s, r)
        lo = jnp.minimum(s, r)
        out.append(_sort_desc(hi))
        carry = _sort_desc(lo)
    return tuple(out)


def _topk_body(x_hbm, out_hbm, rb0, rb1, cand, outv, sem0, sem1):
    wid = lax.axis_index("s") * _NC + lax.axis_index("c")
    iota = lax.iota(jnp.int32, _L)
    ninf = _splat_f32(_NEG_INF)
    row0 = wid * _ROWS_PER_W
    colbases = [jnp.int32(a * _L * _COLCAP) + iota * _COLCAP
                for a in range(_NG)]

    bufs = [rb0, rb1]
    sems = [sem0, sem1]
    copies = [None] * _ROWS_PER_W
    copies[0] = pltpu.async_copy(x_hbm.at[row0], rb0, sem0)

    for i in range(_ROWS_PER_W):
        rowbuf = bufs[i % 2]
        copies[i].wait()
        if i + 1 < _ROWS_PER_W:
            copies[i + 1] = pltpu.async_copy(
                x_hbm.at[row0 + i + 1], bufs[(i + 1) % 2], sems[(i + 1) % 2])

        def p1body(g, b_regs):
            base = g * 256
            vs = [rowbuf[pl.ds(base + u * _L, _L)] for u in range(16)]
            while len(vs) > 1:
                vs = [jnp.maximum(vs[2 * a], vs[2 * a + 1])
                      for a in range(len(vs) // 2)]
            m = vs[0]
            b0, b1, b2, b3 = b_regs
            n0 = jnp.maximum(b0, m)
            r0 = jnp.minimum(b0, m)
            n1 = jnp.maximum(b1, r0)
            r1 = jnp.minimum(b1, r0)
            n2 = jnp.maximum(b2, r1)
            r2 = jnp.minimum(b2, r1)
            n3 = jnp.maximum(b3, r2)
            return (n0, n1, n2, n3)

        b_regs = lax.fori_loop(0, _NVEC // 16, p1body, (ninf,) * 4)
        t = -jnp.max(-b_regs[3])
        t_vec = jnp.full((_L,), t)

        def fbody(g, cnts):
            base = g * 256
            cnts = list(cnts)
            for u in range(16):
                a = u % _NG
                v = rowbuf[pl.ds(base + u * _L, _L)]
                mask = v >= t_vec
                plsc.store_scatter(cand, [colbases[a] + cnts[a]], v,
                                   mask=mask)
                cnts[a] = cnts[a] + mask.astype(jnp.int32)
            return tuple(cnts)

        zero = jnp.zeros((_L,), dtype=jnp.int32)
        cnts = lax.fori_loop(0, _NVEC // 16, fbody, (zero,) * _NG)

        carry = ((ninf, ninf, ninf, ninf), jnp.float32(_NEG_INF))
        for a in range(_NG):
            cnt = cnts[a]
            colbase = colbases[a]
            maxc = jnp.max(cnt)

            def mbody(j, carry, cnt=cnt, colbase=colbase):
                s_regs, t3 = carry
                g = plsc.load_gather(cand, [colbase + j])
                v = jnp.where(cnt > j, g, ninf)
                vm = jnp.max(v)

                def do_merge(c):
                    s_regs, _ = c
                    s_new = _merge_cascade(_sort_desc(v), s_regs)
                    return (s_new, -jnp.max(-s_new[3]))

                return lax.cond(vm > t3, do_merge, lambda c: c,
                                (s_regs, t3))

            carry = lax.fori_loop(0, maxc, mbody, carry)

        s_regs = carry[0]
        for j in range(4):
            outv[pl.ds(j * _L, _L)] = s_regs[j]
        pltpu.sync_copy(outv, out_hbm.at[row0 + i])


@jax.jit
def kernel(x):
    mesh = plsc.VectorSubcoreMesh(core_axis_name="c", subcore_axis_name="s",
                                  num_cores=_NC, num_subcores=_NS)
    return pl.kernel(
        _topk_body,
        out_type=jax.ShapeDtypeStruct((_ROWS, _K), jnp.float32),
        mesh=mesh,
        compiler_params=pltpu.CompilerParams(needs_layout_passes=False),
        scratch_types=[
            pltpu.VMEM((_N,), jnp.float32),
            pltpu.VMEM((_N,), jnp.float32),
            pltpu.VMEM((_NG * _L * _COLCAP,), jnp.float32),
            pltpu.VMEM((_K,), jnp.float32),
            pltpu.SemaphoreType.DMA,
            pltpu.SemaphoreType.DMA,
        ],
    )(x)


# speedup vs baseline: 2.0705x; 2.0705x over previous
import jax
import jax.numpy as jnp
from jax import lax
from jax.experimental import pallas as pl
from jax.experimental.pallas import tpu as pltpu
from jax.experimental.pallas import tpu_sc as plsc

_ROWS = 128
_N = 32768
_K = 64
_NC = 2
_NS = 16
_L = 16
_ROWS_PER_W = _ROWS // (_NC * _NS)

_NVEC = _N // _L
_NG = 4
_COLCAP = _NVEC // _NG + 1

_NEG_INF = float("-inf")


def _splat_f32(v):
    return jnp.full((_L,), v, dtype=jnp.float32)


def _sort_desc(v):
    k, _ = plsc.sort_key_val(v, v, descending=True)
    return k


def _merge_cascade(c_sorted, s_regs):
    out = []
    carry = c_sorted
    for s in s_regs:
        r = lax.rev(carry, (0,))
        hi = jnp.maximum(s, r)
        lo = jnp.minimum(s, r)
        out.append(_sort_desc(hi))
        carry = _sort_desc(lo)
    return tuple(out)


def _topk_body(x_hbm, out_hbm, rb0, rb1, cand, outv, sem0, sem1):
    wid = lax.axis_index("s") * _NC + lax.axis_index("c")
    iota = lax.iota(jnp.int32, _L)
    ninf = _splat_f32(_NEG_INF)
    row0 = wid * _ROWS_PER_W
    colbases = [jnp.int32(a * _L * _COLCAP) + iota * _COLCAP
                for a in range(_NG)]

    bufs = [rb0, rb1]
    sems = [sem0, sem1]
    copies = [None] * _ROWS_PER_W
    copies[0] = pltpu.async_copy(x_hbm.at[row0], rb0, sem0)

    for i in range(_ROWS_PER_W):
        rowbuf = bufs[i % 2]
        copies[i].wait()
        if i + 1 < _ROWS_PER_W:
            copies[i + 1] = pltpu.async_copy(
                x_hbm.at[row0 + i + 1], bufs[(i + 1) % 2], sems[(i + 1) % 2])

        def p1body(g, b_regs):
            base = g * 256
            vs = [rowbuf[pl.ds(base + u * _L, _L)] for u in range(16)]
            while len(vs) > 1:
                vs = [jnp.maximum(vs[2 * a], vs[2 * a + 1])
                      for a in range(len(vs) // 2)]
            m = vs[0]
            b0, b1, b2, b3 = b_regs
            n0 = jnp.maximum(b0, m)
            r0 = jnp.minimum(b0, m)
            n1 = jnp.maximum(b1, r0)
            r1 = jnp.minimum(b1, r0)
            n2 = jnp.maximum(b2, r1)
            r2 = jnp.minimum(b2, r1)
            n3 = jnp.maximum(b3, r2)
            return (n0, n1, n2, n3)

        b_regs = lax.fori_loop(0, _NVEC // 16, p1body, (ninf,) * 4)
        t = -jnp.max(-b_regs[3])
        t_vec = jnp.full((_L,), t)

        zero = jnp.zeros((_L,), dtype=jnp.int32)

        @plsc.parallel_loop(0, _NVEC, step=_NG, unroll=4, carry=(zero,) * _NG)
        def cnts(vi, cnts):
            cnts = list(cnts)
            for a in range(_NG):
                v = rowbuf[pl.ds((vi + a) * _L, _L)]
                mask = v >= t_vec
                plsc.store_scatter(cand, [colbases[a] + cnts[a]], v,
                                   mask=mask)
                cnts[a] = cnts[a] + mask.astype(jnp.int32)
            return tuple(cnts)

        carry = ((ninf, ninf, ninf, ninf), jnp.float32(_NEG_INF))
        for a in range(_NG):
            cnt = cnts[a]
            colbase = colbases[a]
            maxc = jnp.max(cnt)

            def mbody(j, carry, cnt=cnt, colbase=colbase):
                s_regs, t3 = carry
                g = plsc.load_gather(cand, [colbase + j])
                v = jnp.where(cnt > j, g, ninf)
                vm = jnp.max(v)

                def do_merge(c):
                    s_regs, _ = c
                    s_new = _merge_cascade(_sort_desc(v), s_regs)
                    return (s_new, -jnp.max(-s_new[3]))

                return lax.cond(vm > t3, do_merge, lambda c: c,
                                (s_regs, t3))

            carry = lax.fori_loop(0, maxc, mbody, carry)

        s_regs = carry[0]
        for j in range(4):
            outv[pl.ds(j * _L, _L)] = s_regs[j]
        pltpu.sync_copy(outv, out_hbm.at[row0 + i])


@jax.jit
def kernel(x):
    mesh = plsc.VectorSubcoreMesh(core_axis_name="c", subcore_axis_name="s",
                                  num_cores=_NC, num_subcores=_NS)
    return pl.kernel(
        _topk_body,
        out_type=jax.ShapeDtypeStruct((_ROWS, _K), jnp.float32),
        mesh=mesh,
        compiler_params=pltpu.CompilerParams(needs_layout_passes=False),
        scratch_types=[
            pltpu.VMEM((_N,), jnp.float32),
            pltpu.VMEM((_N,), jnp.float32),
            pltpu.VMEM((_NG * _L * _COLCAP,), jnp.float32),
            pltpu.VMEM((_K,), jnp.float32),
            pltpu.SemaphoreType.DMA,
            pltpu.SemaphoreType.DMA,
        ],
    )(x)
